# Initial kernel scaffold; baseline (speedup 1.0000x reference)
import functools

import jax
import jax.numpy as jnp
from jax import lax
from jax.experimental import pallas as pl
from jax.experimental.pallas import tpu as pltpu
from jax.experimental.pallas import tpu_sc as plsc

NSEG = 1024
NC = 2
NS = 16
L = 16
NW = NC * NS


def _seg_sum_kernel(n_atoms: int):
  C = n_atoms // NW
  assert C % L == 0 and (C * 4) % 8 == 0

  mesh = plsc.VectorSubcoreMesh(
      core_axis_name="c", subcore_axis_name="s", num_cores=NC,
      num_subcores=NS)

  @functools.partial(
      pl.kernel,
      out_type=jax.ShapeDtypeStruct((NC, NSEG // L, L), jnp.float32),
      mesh=mesh,
      scratch_types=[
          pltpu.VMEM((C,), jnp.float32),
          pltpu.VMEM((C,), jnp.int32),
          pltpu.VMEM((NSEG // L, L), jnp.float32),
          pltpu.VMEM((NSEG // L,), jnp.int32),
          pltpu.VMEM_SHARED((NSEG // L, L), jnp.float32),
          pltpu.SemaphoreType.DMA,
          pltpu.SemaphoreType.DMA,
      ],
  )
  def kern(energy_hbm, ids_hbm, out_hbm, ev, sv, dense, idx64, shared,
           sem_e, sem_i):
    c = lax.axis_index("c")
    s = lax.axis_index("s")
    wid = c * NS + s
    base = wid * C

    cp_e = pltpu.async_copy(energy_hbm.at[pl.ds(base, C)], ev, sem_e)
    cp_i = pltpu.async_copy(ids_hbm.at[pl.ds(base, C)], sv, sem_i)

    iota = lax.iota(jnp.int32, L)
    zero16 = jnp.zeros((L,), jnp.float32)
    for r in range(NSEG // L):
      dense[r] = zero16
    for r in range(NSEG // (L * L)):
      idx64[pl.ds(r * L, L)] = iota + r * L

    @pl.when(s == 0)
    def _zero_shared():
      pltpu.sync_copy(dense, shared)

    plsc.subcore_barrier()

    cp_i.wait()
    jlo = jnp.min(sv[pl.ds(0, L)])
    jhi = jnp.max(sv[pl.ds(C - L, L)])
    cp_e.wait()

    def search(j):
      lo = jnp.full((L,), -1, jnp.int32)
      hi = jnp.full((L,), C - 1, jnp.int32)
      for _ in range(4):
        step = lax.shift_right_logical(hi - lo + (L - 1), 4)
        pos = jnp.minimum(lo + (iota + 1) * step, hi)
        pos = jnp.clip(pos, 0, C - 1)
        vals = plsc.load_gather(sv, [pos])
        ge = vals >= j
        f = plsc.all_reduce_ffs(ge)
        lo = lo + f * step
        hi = jnp.minimum(lo + step, hi)
      return jnp.max(hi)

    def seg_body(j, p_start):
      p_end = jnp.where(j < jhi, search(j + 1), C)
      k0 = lax.shift_right_arithmetic(p_start, 4)
      k1 = lax.shift_right_arithmetic(p_end - 1, 4) + 1

      def blk(k, acc):
        v = ev[pl.ds(k * L, L)]
        g = k * L + iota
        m = (g >= p_start) & (g < p_end)
        return acc + jnp.where(m, v, 0.0)

      acc = lax.fori_loop(k0, k1, blk, zero16)
      partial = jnp.sum(acc)
      plsc.store_scatter(
          dense,
          [jnp.full((L,), lax.shift_right_arithmetic(j, 4), jnp.int32),
           jnp.full((L,), j & (L - 1), jnp.int32)],
          jnp.full((L,), partial, jnp.float32),
          mask=iota == 0)
      return p_end

    lax.fori_loop(jlo, jhi + 1, seg_body, jnp.int32(0))

    pltpu.sync_copy(dense, shared.at[idx64], add=True)
    plsc.subcore_barrier()

    @pl.when(s == 0)
    def _writeback():
      pltpu.sync_copy(shared, dense)
      pltpu.sync_copy(dense, out_hbm.at[c])

  return kern


def kernel(energy, xyz, segment_ids):
  del xyz
  n = energy.shape[0]
  out2 = _seg_sum_kernel(n)(energy, segment_ids)
  return (out2[0] + out2[1]).reshape(NSEG)


# speedup vs baseline: 40.4361x; 40.4361x over previous
import functools

import jax
import jax.numpy as jnp
from jax import lax
from jax.experimental import pallas as pl
from jax.experimental.pallas import tpu as pltpu
from jax.experimental.pallas import tpu_sc as plsc

NSEG = 1024
NC = 2
NS = 16
L = 16
NW = NC * NS


def _seg_sum_kernel(n_atoms: int):
  C = n_atoms // NW
  assert C % L == 0 and (C * 4) % 8 == 0

  mesh = plsc.VectorSubcoreMesh(
      core_axis_name="c", subcore_axis_name="s", num_cores=NC,
      num_subcores=NS)

  @functools.partial(
      pl.kernel,
      out_type=jax.ShapeDtypeStruct((NC, NSEG // L, L), jnp.float32),
      mesh=mesh,
      compiler_params=pltpu.CompilerParams(needs_layout_passes=False),
      scratch_types=[
          pltpu.VMEM((C,), jnp.float32),
          pltpu.VMEM((C,), jnp.int32),
          pltpu.VMEM((NSEG // L, L), jnp.float32),
          pltpu.VMEM((NSEG // L // NS, L), jnp.float32),
          pltpu.VMEM((NSEG // L // NS, L), jnp.float32),
          pltpu.VMEM_SHARED((NS, NSEG // L, L), jnp.float32),
          pltpu.SemaphoreType.DMA,
          pltpu.SemaphoreType.DMA,
      ],
  )
  def kern(energy_hbm, ids_hbm, out_hbm, ev, sv, dense, accbuf, buf4,
           shared, sem_e, sem_i):
    c = lax.axis_index("c")
    s = lax.axis_index("s")
    wid = c * NS + s
    base = wid * C

    cp_e = pltpu.async_copy(energy_hbm.at[pl.ds(base, C)], ev, sem_e)
    cp_i = pltpu.async_copy(ids_hbm.at[pl.ds(base, C)], sv, sem_i)

    iota = lax.iota(jnp.int32, L)
    zero16 = jnp.zeros((L,), jnp.float32)
    for r in range(NSEG // L):
      dense[r] = zero16

    cp_i.wait()
    jlo = jnp.min(sv[pl.ds(0, L)])
    jhi = jnp.max(sv[pl.ds(C - L, L)])
    cp_e.wait()

    def search(j):
      lo = jnp.full((L,), -1, jnp.int32)
      hi = jnp.full((L,), C - 1, jnp.int32)
      for _ in range(4):
        step = lax.shift_right_logical(hi - lo + (L - 1), 4)
        pos = jnp.minimum(lo + (iota + 1) * step, hi)
        pos = jnp.clip(pos, 0, C - 1)
        vals = plsc.load_gather(sv, [pos])
        ge = vals >= j
        f = plsc.all_reduce_ffs(ge)
        lo = lo + f * step
        hi = jnp.minimum(lo + step, hi)
      return jnp.max(hi)

    def seg_body(j, p_start):
      p_end = jnp.where(j < jhi, search(j + 1), C)
      k0 = lax.shift_right_arithmetic(p_start, 4)
      k1 = lax.shift_right_arithmetic(p_end - 1, 4) + 1

      def blk(k, acc):
        v = ev[pl.ds(k * L, L)]
        g = k * L + iota
        m = (g >= p_start) & (g < p_end)
        return acc + jnp.where(m, v, 0.0)

      acc = lax.fori_loop(k0, k1, blk, zero16)
      partial = jnp.sum(acc)
      plsc.store_scatter(
          dense,
          [jnp.full((L,), lax.shift_right_arithmetic(j, 4), jnp.int32),
           jnp.full((L,), j & (L - 1), jnp.int32)],
          jnp.full((L,), partial, jnp.float32),
          mask=iota == 0)
      return p_end

    lax.fori_loop(jlo, jhi + 1, seg_body, jnp.int32(0))

    RPT = NSEG // L // NS
    pltpu.sync_copy(dense, shared.at[s])
    plsc.subcore_barrier()
    for r in range(RPT):
      accbuf[r] = zero16
    for w in range(NS):
      pltpu.sync_copy(shared.at[w, pl.ds(s * RPT, RPT)], buf4)
      for r in range(RPT):
        accbuf[r] = accbuf[r] + buf4[r]
    pltpu.sync_copy(accbuf, out_hbm.at[c, pl.ds(s * RPT, RPT)])

  return kern


def kernel(energy, xyz, segment_ids):
  del xyz
  n = energy.shape[0]
  out2 = _seg_sum_kernel(n)(energy, segment_ids)
  return (out2[0] + out2[1]).reshape(NSEG)


# speedup vs baseline: 47.4984x; 1.1747x over previous
import functools

import jax
import jax.numpy as jnp
from jax import lax
from jax.experimental import pallas as pl
from jax.experimental.pallas import tpu as pltpu
from jax.experimental.pallas import tpu_sc as plsc

NSEG = 1024
NC = 2
NS = 16
L = 16
NW = NC * NS


def _seg_sum_kernel(n_atoms: int):
  C = n_atoms // NW
  assert C % L == 0 and (C * 4) % 8 == 0

  mesh = plsc.VectorSubcoreMesh(
      core_axis_name="c", subcore_axis_name="s", num_cores=NC,
      num_subcores=NS)

  @functools.partial(
      pl.kernel,
      out_type=jax.ShapeDtypeStruct((NC, NSEG // L, L), jnp.float32),
      mesh=mesh,
      compiler_params=pltpu.CompilerParams(needs_layout_passes=False),
      scratch_types=[
          pltpu.VMEM((C,), jnp.float32),
          pltpu.VMEM((C,), jnp.int32),
          pltpu.VMEM((NSEG // L, L), jnp.float32),
          pltpu.VMEM((NSEG // L // NS, L), jnp.float32),
          pltpu.VMEM((NSEG // L // NS, L), jnp.float32),
          pltpu.VMEM_SHARED((NS, NSEG // L, L), jnp.float32),
          pltpu.SemaphoreType.DMA,
          pltpu.SemaphoreType.DMA,
      ],
  )
  def kern(energy_hbm, ids_hbm, out_hbm, ev, sv, dense, accbuf, buf4,
           shared, sem_e, sem_i):
    c = lax.axis_index("c")
    s = lax.axis_index("s")
    wid = c * NS + s
    base = wid * C

    cp_e = pltpu.async_copy(energy_hbm.at[pl.ds(base, C)], ev, sem_e)
    cp_i = pltpu.async_copy(ids_hbm.at[pl.ds(base, C)], sv, sem_i)

    iota = lax.iota(jnp.int32, L)
    zero16 = jnp.zeros((L,), jnp.float32)
    for r in range(NSEG // L):
      dense[r] = zero16

    cp_i.wait()
    jlo = jnp.min(sv[pl.ds(0, L)])
    jhi = jnp.max(sv[pl.ds(C - L, L)])
    cp_e.wait()

    def search(j):
      lo = jnp.full((L,), -1, jnp.int32)
      hi = jnp.full((L,), C - 1, jnp.int32)
      for _ in range(4):
        step = lax.shift_right_logical(hi - lo + (L - 1), 4)
        pos = jnp.minimum(lo + (iota + 1) * step, hi)
        pos = jnp.clip(pos, 0, C - 1)
        vals = plsc.load_gather(sv, [pos])
        ge = vals >= j
        f = plsc.all_reduce_ffs(ge)
        lo = lo + f * step
        hi = jnp.minimum(lo + step, hi)
      return jnp.max(hi)

    def seg_body(j, p_start):
      p_end = jnp.where(j < jhi, search(j + 1), C)
      ka = lax.shift_right_arithmetic(p_start, 4)
      kb = lax.shift_right_arithmetic(jnp.maximum(p_end - 1, 0), 4)
      gh = ka * L + iota
      head = jnp.where((gh >= p_start) & (gh < p_end),
                       ev[pl.ds(ka * L, L)], 0.0)
      gt = kb * L + iota
      tail = jnp.where((gt >= p_start) & (gt < p_end) & (ka < kb),
                       ev[pl.ds(kb * L, L)], 0.0)

      @plsc.parallel_loop(ka + 1, kb, unroll=8, carry=head + tail)
      def acc(k, a):
        return a + ev[pl.ds(k * L, L)]

      partial = jnp.sum(acc)
      plsc.store_scatter(
          dense,
          [jnp.full((L,), lax.shift_right_arithmetic(j, 4), jnp.int32),
           jnp.full((L,), j & (L - 1), jnp.int32)],
          jnp.full((L,), partial, jnp.float32),
          mask=iota == 0)
      return p_end

    lax.fori_loop(jlo, jhi + 1, seg_body, jnp.int32(0))

    RPT = NSEG // L // NS
    pltpu.sync_copy(dense, shared.at[s])
    plsc.subcore_barrier()
    for r in range(RPT):
      accbuf[r] = zero16
    for w in range(NS):
      pltpu.sync_copy(shared.at[w, pl.ds(s * RPT, RPT)], buf4)
      for r in range(RPT):
        accbuf[r] = accbuf[r] + buf4[r]
    pltpu.sync_copy(accbuf, out_hbm.at[c, pl.ds(s * RPT, RPT)])

  return kern


def kernel(energy, xyz, segment_ids):
  del xyz
  n = energy.shape[0]
  out2 = _seg_sum_kernel(n)(energy, segment_ids)
  return (out2[0] + out2[1]).reshape(NSEG)
